# Initial kernel scaffold; baseline (speedup 1.0000x reference)
import jax
import jax.numpy as jnp
from jax.experimental import pallas as pl


def kernel(x, edge_index, edge_attr, init_w1, root_w1, bias1, init_w2, root_w2, bias2):
    raise NotImplementedError("write your pallas kernel here")


# speedup vs baseline: 14.9251x; 14.9251x over previous
import functools

import jax
import jax.numpy as jnp
from jax import lax
from jax.experimental import pallas as pl
from jax.experimental.pallas import tpu as pltpu
from jax.experimental.pallas import tpu_sc as plsc

N = 10000
E = 320000
F_IN = 128
HID = 64
NCLS = 16

NC = 2
NS = 16
NW = NC * NS
EPW = E // NW
CHUNK = 80
NCH = EPW // CHUNK
RPT = N // NS


def _prop_body(F, src_hbm, dst_hbm, ea_hbm, feat_hbm, zero_hbm, out_hbm,
               src_all, dst_all, ea_all, rows_v, acc, sem):
    cid = lax.axis_index("c")
    sid = lax.axis_index("s")
    wid = sid * NC + cid

    pltpu.sync_copy(zero_hbm.at[pl.ds(sid * RPT, RPT)],
                    acc.at[pl.ds(sid * RPT, RPT)])
    pltpu.sync_copy(src_hbm.at[wid], src_all)
    pltpu.sync_copy(dst_hbm.at[wid], dst_all)
    pltpu.sync_copy(ea_hbm.at[wid], ea_all)
    plsc.subcore_barrier()

    def chunk_body(j, carry):
        src_j = src_all.at[j]
        dst_j = dst_all.at[j]
        pltpu.async_copy(feat_hbm.at[src_j], rows_v, sem).wait()

        def group_body(g, c2):
            eav = ea_all[j, pl.ds(g * 16, 16)]
            for e16 in range(16):
                sv = jnp.full((16,), eav[e16], jnp.float32)
                e = g * 16 + e16
                for fb in range(F // 16):
                    sl = pl.ds(fb * 16, 16)
                    rows_v[e, sl] = rows_v[e, sl] * sv
            return c2

        lax.fori_loop(0, CHUNK // 16, group_body, 0)
        pltpu.sync_copy(rows_v, acc.at[dst_j], add=True)
        return carry

    lax.fori_loop(0, NCH, chunk_body, 0)
    plsc.subcore_barrier()
    pltpu.sync_copy(acc.at[pl.ds(sid * RPT, RPT)],
                    out_hbm.at[cid, pl.ds(sid * RPT, RPT)])


def _make_prop(F):
    mesh = plsc.VectorSubcoreMesh(core_axis_name="c", subcore_axis_name="s")
    return functools.partial(
        pl.kernel,
        out_type=jax.ShapeDtypeStruct((NC, N, F), jnp.float32),
        mesh=mesh,
        scratch_types=[
            pltpu.VMEM((NCH, CHUNK), jnp.int32),
            pltpu.VMEM((NCH, CHUNK), jnp.int32),
            pltpu.VMEM((NCH, CHUNK), jnp.float32),
            pltpu.VMEM((CHUNK, F), jnp.float32),
            pltpu.VMEM_SHARED((N, F), jnp.float32),
            pltpu.SemaphoreType.DMA,
        ],
        compiler_params=pltpu.CompilerParams(use_tc_tiling_on_sc=False),
    )(functools.partial(_prop_body, F))


def _deg_body(dst_hbm, ea_hbm, zero_hbm, out_hbm, dst_all, ea_all, rows_v,
              acc, sem):
    cid = lax.axis_index("c")
    sid = lax.axis_index("s")
    wid = sid * NC + cid

    pltpu.sync_copy(zero_hbm.at[pl.ds(sid * RPT, RPT)],
                    acc.at[pl.ds(sid * RPT, RPT)])
    pltpu.sync_copy(dst_hbm.at[wid], dst_all)
    pltpu.sync_copy(ea_hbm.at[wid], ea_all)
    plsc.subcore_barrier()

    def chunk_body(j, carry):
        dst_j = dst_all.at[j]

        def group_body(g, c2):
            eav = ea_all[j, pl.ds(g * 16, 16)]
            for e16 in range(16):
                rows_v[g * 16 + e16, :] = jnp.full((16,), eav[e16], jnp.float32)
            return c2

        lax.fori_loop(0, CHUNK // 16, group_body, 0)
        pltpu.sync_copy(rows_v, acc.at[dst_j], add=True)
        return carry

    lax.fori_loop(0, NCH, chunk_body, 0)
    plsc.subcore_barrier()

    pltpu.sync_copy(acc.at[pl.ds(sid * RPT, RPT)],
                    out_hbm.at[cid, pl.ds(sid * RPT, RPT)])


_deg_kernel = functools.partial(
    pl.kernel,
    out_type=jax.ShapeDtypeStruct((NC, N, 16), jnp.float32),
    mesh=plsc.VectorSubcoreMesh(core_axis_name="c", subcore_axis_name="s"),
    scratch_types=[
        pltpu.VMEM((NCH, CHUNK), jnp.int32),
        pltpu.VMEM((NCH, CHUNK), jnp.float32),
        pltpu.VMEM((CHUNK, 16), jnp.float32),
        pltpu.VMEM_SHARED((N, 16), jnp.float32),
        pltpu.SemaphoreType.DMA,
    ],
    compiler_params=pltpu.CompilerParams(use_tc_tiling_on_sc=False),
)(_deg_body)


def _tc1_body(degp_ref, x_ref, w1_ref, r1_ref, dis_ref, feat1_ref, root1_ref):
    deg = (degp_ref[0] + degp_ref[1])[:, 0:1]
    dis = jnp.where(deg > 0, lax.rsqrt(jnp.maximum(deg, 1e-12)), 0.0)
    dis_ref[...] = dis
    xw = jnp.dot(x_ref[...], w1_ref[...], preferred_element_type=jnp.float32)
    feat1_ref[...] = xw * dis
    root1_ref[...] = jnp.dot(x_ref[...], r1_ref[...],
                             preferred_element_type=jnp.float32)


def _tc2_body(t1p_ref, dis_ref, root1_ref, b1_ref, w2_ref, r2_ref,
              feat2_ref, root2_ref):
    dis = dis_ref[...]
    agg = (t1p_ref[0] + t1p_ref[1]) * dis
    h = jax.nn.relu(agg + root1_ref[...] + b1_ref[...])
    feat2_ref[...] = jnp.dot(h, w2_ref[...],
                             preferred_element_type=jnp.float32) * dis
    root2_ref[...] = jnp.dot(h, r2_ref[...],
                             preferred_element_type=jnp.float32)


def _tc3_body(t2p_ref, dis_ref, root2_ref, b2_ref, out_ref):
    agg = (t2p_ref[0] + t2p_ref[1]) * dis_ref[...]
    o = jax.nn.relu(agg + root2_ref[...] + b2_ref[...])
    m = jnp.max(o, axis=1, keepdims=True)
    z = o - m
    out_ref[...] = z - jnp.log(jnp.sum(jnp.exp(z), axis=1, keepdims=True))


def kernel(x, edge_index, edge_attr, init_w1, root_w1, bias1,
           init_w2, root_w2, bias2):
    src = edge_index[0].reshape(NW, NCH, CHUNK)
    dst = edge_index[1].reshape(NW, NCH, CHUNK)
    ea = edge_attr.reshape(NW, NCH, CHUNK)
    w1, r1, b1 = init_w1[0], root_w1[0], bias1[0].reshape(1, HID)
    w2, r2, b2 = init_w2[0], root_w2[0], bias2[0].reshape(1, NCLS)
    zeros64 = jnp.zeros((N, HID), jnp.float32)
    zeros16 = jnp.zeros((N, NCLS), jnp.float32)

    degp = _deg_kernel(dst, ea, zeros16)

    dis, feat1, root1 = pl.pallas_call(
        _tc1_body,
        out_shape=(jax.ShapeDtypeStruct((N, 1), jnp.float32),
                   jax.ShapeDtypeStruct((N, HID), jnp.float32),
                   jax.ShapeDtypeStruct((N, HID), jnp.float32)),
    )(degp, x, w1, r1)

    t1p = _make_prop(HID)(src, dst, ea, feat1, zeros64)

    feat2, root2 = pl.pallas_call(
        _tc2_body,
        out_shape=(jax.ShapeDtypeStruct((N, NCLS), jnp.float32),
                   jax.ShapeDtypeStruct((N, NCLS), jnp.float32)),
    )(t1p, dis, root1, b1, w2, r2)

    t2p = _make_prop(NCLS)(src, dst, ea, feat2, zeros16)

    out = pl.pallas_call(
        _tc3_body,
        out_shape=jax.ShapeDtypeStruct((N, NCLS), jnp.float32),
    )(t2p, dis, root2, b2)
    return out
